# baseline (speedup 1.0000x reference)
import jax
import jax.numpy as jnp
from jax import lax
from jax.experimental import pallas as pl
from jax.experimental.pallas import tpu as pltpu
from jax.experimental.pallas import tpu_sc as plsc

N, H, W, K, F, D = 8, 224, 224, 1, 100000, 3
HW = H * W
P = N * HW
NW = 32
PW = P // NW
CHUNK = 1792
NCHUNK = PW // CHUNK
NVEC = CHUNK // 16
NREC = N * F
VROWS = NREC * 9 // 8
RPW = NREC // NW
RCH = 1000
NRCH = RPW // RCH
ORROWS = RCH * 9 // 8

def _detile_body(tv_hbm, tr_hbm, inbuf, outbuf):
    wid = lax.axis_index("c") * 16 + lax.axis_index("s")
    iota9 = 9 * lax.iota(jnp.int32, 16)
    for c in range(NRCH):
        base = wid * RPW + c * RCH
        pltpu.sync_copy(tv_hbm.at[:, pl.ds(base, RCH)],
                        inbuf.at[:, pl.ds(0, RCH)])
        def sc_body(j, _):
            w0 = 144 * j + iota9
            for p in range(9):
                w = w0 + p
                plsc.store_scatter(outbuf, [w >> 3, w & 7],
                                   inbuf[p, pl.ds(16 * j, 16)])
            return 0
        lax.fori_loop(0, 63, sc_body, 0)
        pltpu.sync_copy(outbuf.at[pl.ds(0, ORROWS)],
                        tr_hbm.at[pl.ds(wid * (RPW * 9 // 8) + c * ORROWS,
                                        ORROWS)])


def _raster_body(attr_hbm, bary_hbm, p2f_hbm, out_hbm,
                 idx_raw, idx2, bary_v, rows_v, acc_v, sem):
    wid = lax.axis_index("c") * 16 + lax.axis_index("s")
    img = wid // 4
    quad = wid - img * 4
    iota = lax.iota(jnp.int32, 16)

    for c in range(NCHUNK):
        pbase = wid * PW + c * CHUNK
        pltpu.sync_copy(p2f_hbm.at[pl.ds(pbase, CHUNK)], idx_raw)

        for s in range(4):
            srow = jnp.full((16,), s, jnp.int32)
            def idx_body(i, _, s=s, srow=srow):
                raw = idx_raw[pl.ds(s * 448 + i * 16, 16)]
                a = (9 * jnp.maximum(raw, 0)) >> 3
                pos = 32 * i + 2 * iota
                plsc.store_scatter(idx2, [srow, pos], a)
                plsc.store_scatter(idx2, [srow, pos + 1], a + 1)
                return 0
            lax.fori_loop(0, 28, idx_body, 0)

        cps = [pltpu.async_copy(attr_hbm.at[idx2.at[s]],
                                rows_v.at[pl.ds(s * 896, 896)], sem)
               for s in range(4)]
        pltpu.sync_copy(bary_hbm.at[pl.ds(wid * 56 + c * 8, 8), :], bary_v)
        for cp in cps:
            cp.wait()

        def row_body(g, _):
            def vec_body(j, _):
                pix = g * 224 + 16 * j
                raw = idx_raw[pl.ds(pix, 16)]
                vis = jnp.where(raw < 0, 0.0, 1.0)
                base = 16 * (pix + iota) + ((9 * jnp.maximum(raw, 0)) & 7)
                r = []
                for cc in range(9):
                    t = base + cc
                    r.append(plsc.load_gather(rows_v, [t >> 3, t & 7]))
                b = [bary_v[g, pl.ds(v * 224 + 16 * j, 16)] for v in range(3)]
                for d in range(3):
                    acc_v[d, pl.ds(pix, 16)] = vis * (b[0] * r[d]
                                                      + b[1] * r[3 + d]
                                                      + b[2] * r[6 + d])
                acc_v[3, pl.ds(pix, 16)] = vis
                return 0
            lax.fori_loop(0, 14, vec_body, 0)
            return 0
        lax.fori_loop(0, 8, row_body, 0)

        obase = img * (4 * HW) + quad * PW + c * CHUNK
        for ch in range(4):
            pltpu.sync_copy(acc_v.at[ch], out_hbm.at[pl.ds(obase + ch * HW,
                                                           CHUNK)])


def _sc_mesh():
    return plsc.VectorSubcoreMesh(core_axis_name="c", subcore_axis_name="s")


_CP = pltpu.CompilerParams(use_tc_tiling_on_sc=False,
                           needs_layout_passes=False)


@jax.jit
def _run(attrs, bary_coords, pix_to_face):
    tv = jnp.transpose(attrs, (2, 3, 0, 1)).reshape(9, NREC)
    baryn = jnp.transpose(bary_coords, (0, 1, 4, 3, 2)).reshape(N * H, 3 * W)
    p2f = pix_to_face.reshape(P)

    tr = pl.kernel(
        _detile_body,
        out_type=jax.ShapeDtypeStruct((VROWS, 8), jnp.float32),
        mesh=_sc_mesh(),
        compiler_params=_CP,
        scratch_types=[
            pltpu.VMEM((9, RCH + 8), jnp.float32),
            pltpu.VMEM((ORROWS + 27, 8), jnp.float32),
        ],
    )(tv)

    out = pl.kernel(
        _raster_body,
        out_type=jax.ShapeDtypeStruct((N * 4 * HW,), jnp.float32),
        mesh=_sc_mesh(),
        compiler_params=_CP,
        scratch_types=[
            pltpu.VMEM((CHUNK,), jnp.int32),
            pltpu.VMEM((4, 896), jnp.int32),
            pltpu.VMEM((8, 3 * W), jnp.float32),
            pltpu.VMEM((2 * CHUNK, 8), jnp.float32),
            pltpu.VMEM((4, CHUNK), jnp.float32),
            pltpu.SemaphoreType.DMA,
        ],
    )(tr, baryn, p2f)
    return out.reshape(N, 4, H, W)


def kernel(attributes, bary_coords, pix_to_face):
    return _run(attributes, bary_coords, pix_to_face)


# speedup vs baseline: 1.7594x; 1.7594x over previous
import jax
import jax.numpy as jnp
from jax import lax
from jax.experimental import pallas as pl
from jax.experimental.pallas import tpu as pltpu
from jax.experimental.pallas import tpu_sc as plsc

N, H, W, K, F, D = 8, 224, 224, 1, 100000, 3
HW = H * W
P = N * HW
NW = 32
PW = P // NW
CHUNK = 1792
NCHUNK = PW // CHUNK
NVEC = CHUNK // 16
NREC = N * F
RPW = NREC // NW
RCH = 1000
NRCH = RPW // RCH
NSTR = 4
SLEN = CHUNK // NSTR


def _detile_body(tv_hbm, tr_hbm, inbuf, outbuf):
    wid = lax.axis_index("c") * 16 + lax.axis_index("s")
    iota = lax.iota(jnp.int32, 16)
    for c in range(NRCH):
        base = wid * RPW + c * RCH
        pltpu.sync_copy(tv_hbm.at[:, pl.ds(base, RCH)],
                        inbuf.at[:, pl.ds(0, RCH)])
        def sc_body(j, _):
            rec = 16 * j + iota
            for p in range(9):
                plsc.store_scatter(outbuf, [rec, jnp.full((16,), p, jnp.int32)],
                                   inbuf[p, pl.ds(16 * j, 16)])
            return 0
        lax.fori_loop(0, 63, sc_body, 0)
        pltpu.sync_copy(outbuf.at[pl.ds(0, RCH)],
                        tr_hbm.at[pl.ds(base, RCH)])


def _raster_body(attr_hbm, bary_hbm, p2f_hbm, out_hbm,
                 idx_raw, idx2, bary_v, rows_v, acc_v, sem):
    wid = lax.axis_index("c") * 16 + lax.axis_index("s")
    img = wid // 4
    quad = wid - img * 4
    iota = lax.iota(jnp.int32, 16)

    for c in range(NCHUNK):
        pbase = wid * PW + c * CHUNK
        pltpu.sync_copy(p2f_hbm.at[pl.ds(pbase, CHUNK)], idx_raw)

        for s in range(NSTR):
            def idx_body(i, _, s=s):
                raw = idx_raw[pl.ds(s * SLEN + i * 16, 16)]
                idx2[s, pl.ds(i * 16, 16)] = jnp.maximum(raw, 0)
                return 0
            lax.fori_loop(0, SLEN // 16, idx_body, 0)

        cps = [pltpu.async_copy(attr_hbm.at[idx2.at[s]],
                                rows_v.at[pl.ds(s * SLEN, SLEN)], sem)
               for s in range(NSTR)]
        pltpu.sync_copy(bary_hbm.at[pl.ds(wid * 56 + c * 8, 8), :], bary_v)
        for cp in cps:
            cp.wait()

        def row_body(g, _):
            def vec_body(j, _):
                pix = g * 224 + 16 * j
                raw = idx_raw[pl.ds(pix, 16)]
                vis = jnp.where(raw < 0, 0.0, 1.0)
                pv = pix + iota
                r = [plsc.load_gather(rows_v, [pv, jnp.full((16,), cc, jnp.int32)])
                     for cc in range(9)]
                b = [bary_v[g, pl.ds(v * 224 + 16 * j, 16)] for v in range(3)]
                for d in range(3):
                    acc_v[d, pl.ds(pix, 16)] = vis * (b[0] * r[d]
                                                      + b[1] * r[3 + d]
                                                      + b[2] * r[6 + d])
                acc_v[3, pl.ds(pix, 16)] = vis
                return 0
            lax.fori_loop(0, 14, vec_body, 0)
            return 0
        lax.fori_loop(0, 8, row_body, 0)

        obase = img * (4 * HW) + quad * PW + c * CHUNK
        for ch in range(4):
            pltpu.sync_copy(acc_v.at[ch], out_hbm.at[pl.ds(obase + ch * HW,
                                                           CHUNK)])


def _sc_mesh():
    return plsc.VectorSubcoreMesh(core_axis_name="c", subcore_axis_name="s")


_CP = pltpu.CompilerParams(use_tc_tiling_on_sc=False,
                           needs_layout_passes=False)


@jax.jit
def _run(attrs, bary_coords, pix_to_face):
    tv = jnp.transpose(attrs, (2, 3, 0, 1)).reshape(9, NREC)
    baryn = jnp.transpose(bary_coords, (0, 1, 4, 3, 2)).reshape(N * H, 3 * W)
    p2f = pix_to_face.reshape(P)

    tr = pl.kernel(
        _detile_body,
        out_type=jax.ShapeDtypeStruct((NREC, 16), jnp.float32),
        mesh=_sc_mesh(),
        compiler_params=_CP,
        scratch_types=[
            pltpu.VMEM((9, RCH + 8), jnp.float32),
            pltpu.VMEM((RCH + 8, 16), jnp.float32),
        ],
    )(tv)

    out = pl.kernel(
        _raster_body,
        out_type=jax.ShapeDtypeStruct((N * 4 * HW,), jnp.float32),
        mesh=_sc_mesh(),
        compiler_params=_CP,
        scratch_types=[
            pltpu.VMEM((CHUNK,), jnp.int32),
            pltpu.VMEM((NSTR, SLEN), jnp.int32),
            pltpu.VMEM((8, 3 * W), jnp.float32),
            pltpu.VMEM((CHUNK, 16), jnp.float32),
            pltpu.VMEM((4, CHUNK), jnp.float32),
            pltpu.SemaphoreType.DMA,
        ],
    )(tr, baryn, p2f)
    return out.reshape(N, 4, H, W)


def kernel(attributes, bary_coords, pix_to_face):
    return _run(attributes, bary_coords, pix_to_face)


# speedup vs baseline: 4.9912x; 2.8368x over previous
import jax
import jax.numpy as jnp
from jax import lax
from jax.experimental import pallas as pl
from jax.experimental.pallas import tpu as pltpu
from jax.experimental.pallas import tpu_sc as plsc

N, H, W, K, F, D = 8, 224, 224, 1, 100000, 3
HW = H * W
P = N * HW
NW = 32
PW = P // NW
CHUNK = 1792
NCHUNK = PW // CHUNK
NVEC = CHUNK // 16
NREC = N * F
RPW = NREC // NW
RCH = 1000
NRCH = RPW // RCH
NSTR = 7
SLEN = 256


def _detile_body(tv_hbm, tr_hbm, inbuf, outbuf):
    wid = lax.axis_index("c") * 16 + lax.axis_index("s")
    iota = lax.iota(jnp.int32, 16)
    for c in range(NRCH):
        base = wid * RPW + c * RCH
        pltpu.sync_copy(tv_hbm.at[:, pl.ds(base, RCH)],
                        inbuf.at[:, pl.ds(0, RCH)])
        def sc_body(j, _):
            rec = 16 * j + iota
            for p in range(9):
                plsc.store_scatter(outbuf, [rec, jnp.full((16,), p, jnp.int32)],
                                   inbuf[p, pl.ds(16 * j, 16)])
            return 0
        lax.fori_loop(0, 63, sc_body, 0)
        pltpu.sync_copy(outbuf.at[pl.ds(0, RCH)],
                        tr_hbm.at[pl.ds(base, RCH)])


def _raster_body(attr_hbm, bary_hbm, p2f_hbm, out_hbm,
                 idx_raw, idx2, rowix, bary_v, rows_v, acc_v, sem):
    wid = lax.axis_index("c") * 16 + lax.axis_index("s")
    img = wid // 4
    quad = wid - img * 4
    iota = lax.iota(jnp.int32, 16)

    def zero_body(i, _):
        idx2[pl.ds(i * 16, 16)] = jnp.zeros((16,), jnp.int32)
        return 0
    lax.fori_loop(0, (CHUNK + 16) // 16, zero_body, 0)

    for c in range(NCHUNK):
        pbase = wid * PW + c * CHUNK
        pltpu.sync_copy(p2f_hbm.at[pl.ds(pbase, CHUNK)], idx_raw)

        def idx_body(i, off):
            raw = idx_raw[pl.ds(i * 16, 16)]
            m = raw > -1
            mi = m.astype(jnp.int32)
            cs = plsc.cumsum(mi)
            rowix[pl.ds(i * 16, 16)] = jnp.maximum(off + cs - 1, 0)
            plsc.store_compressed(idx2.at[pl.ds(off, 16)],
                                  jnp.maximum(raw, 0), mask=m)
            return off + jnp.sum(mi)
        nvis = lax.fori_loop(0, NVEC, idx_body, 0)

        nstr = (nvis + (SLEN - 1)) >> 8
        for s in range(NSTR):
            @pl.when(s < nstr)
            def _(s=s):
                pltpu.async_copy(attr_hbm.at[idx2.at[pl.ds(s * SLEN, SLEN)]],
                                 rows_v.at[pl.ds(s * SLEN, SLEN)], sem)
        pltpu.sync_copy(bary_hbm.at[pl.ds(wid * 56 + c * 8, 8), :], bary_v)
        for s in range(NSTR):
            @pl.when(s < nstr)
            def _(s=s):
                pltpu.make_async_copy(
                    attr_hbm.at[idx2.at[pl.ds(s * SLEN, SLEN)]],
                    rows_v.at[pl.ds(s * SLEN, SLEN)], sem).wait()

        def row_body(g, _):
            def vec_body(j, _):
                pix = g * 224 + 16 * j
                raw = idx_raw[pl.ds(pix, 16)]
                vis = jnp.where(raw < 0, 0.0, 1.0)
                rv = rowix[pl.ds(pix, 16)]
                r = [plsc.load_gather(rows_v, [rv, jnp.full((16,), cc, jnp.int32)])
                     for cc in range(9)]
                b = [bary_v[g, pl.ds(v * 224 + 16 * j, 16)] for v in range(3)]
                for d in range(3):
                    acc_v[d, pl.ds(pix, 16)] = vis * (b[0] * r[d]
                                                      + b[1] * r[3 + d]
                                                      + b[2] * r[6 + d])
                acc_v[3, pl.ds(pix, 16)] = vis
                return 0
            lax.fori_loop(0, 14, vec_body, 0)
            return 0
        lax.fori_loop(0, 8, row_body, 0)

        obase = img * (4 * HW) + quad * PW + c * CHUNK
        for ch in range(4):
            pltpu.sync_copy(acc_v.at[ch], out_hbm.at[pl.ds(obase + ch * HW,
                                                           CHUNK)])


def _sc_mesh():
    return plsc.VectorSubcoreMesh(core_axis_name="c", subcore_axis_name="s")


_CP = pltpu.CompilerParams(use_tc_tiling_on_sc=False,
                           needs_layout_passes=False)


@jax.jit
def _run(attrs, bary_coords, pix_to_face):
    tv = jnp.transpose(attrs, (2, 3, 0, 1)).reshape(9, NREC)
    baryn = jnp.transpose(bary_coords, (0, 1, 4, 3, 2)).reshape(N * H, 3 * W)
    p2f = pix_to_face.reshape(P)

    tr = pl.kernel(
        _detile_body,
        out_type=jax.ShapeDtypeStruct((NREC, 16), jnp.float32),
        mesh=_sc_mesh(),
        compiler_params=_CP,
        scratch_types=[
            pltpu.VMEM((9, RCH + 8), jnp.float32),
            pltpu.VMEM((RCH + 8, 16), jnp.float32),
        ],
    )(tv)

    out = pl.kernel(
        _raster_body,
        out_type=jax.ShapeDtypeStruct((N * 4 * HW,), jnp.float32),
        mesh=_sc_mesh(),
        compiler_params=_CP,
        scratch_types=[
            pltpu.VMEM((CHUNK,), jnp.int32),
            pltpu.VMEM((CHUNK + 16,), jnp.int32),
            pltpu.VMEM((CHUNK,), jnp.int32),
            pltpu.VMEM((8, 3 * W), jnp.float32),
            pltpu.VMEM((CHUNK, 16), jnp.float32),
            pltpu.VMEM((4, CHUNK), jnp.float32),
            pltpu.SemaphoreType.DMA,
        ],
    )(tr, baryn, p2f)
    return out.reshape(N, 4, H, W)


def kernel(attributes, bary_coords, pix_to_face):
    return _run(attributes, bary_coords, pix_to_face)
